# Initial kernel scaffold; baseline (speedup 1.0000x reference)
import jax
import jax.numpy as jnp
from jax.experimental import pallas as pl


def kernel(x, edge_index, W_self0, W_neigh0, b0, W_self1, W_neigh1, b1):
    raise NotImplementedError("write your pallas kernel here")


# speedup vs baseline: 7.1258x; 7.1258x over previous
import functools

import jax
import jax.numpy as jnp
from jax import lax
from jax.experimental import pallas as pl
from jax.experimental.pallas import tpu as pltpu
from jax.experimental.pallas import tpu_sc as plsc

N = 10000
E = 320000
D_IN = 128
D_HID = 256
D_OUT = 128

NC = 2
NS = 16
CHUNK = 128
NP = 10240
ROWS_PER_TILE = NP // NS
ZR = 128
N_COPIES = ROWS_PER_TILE // ZR
CHUNKS_PER_CORE = E // (NC * CHUNK)
ITERS = (CHUNKS_PER_CORE + NS - 1) // NS


def _make_segsum(D):
    mesh = plsc.VectorSubcoreMesh(core_axis_name="c", subcore_axis_name="s")

    @functools.partial(
        pl.kernel,
        mesh=mesh,
        compiler_params=pltpu.CompilerParams(use_tc_tiling_on_sc=False),
        out_type=jax.ShapeDtypeStruct((NC * NP, D), jnp.float32),
        scratch_types=[
            pltpu.VMEM_SHARED((NP, D), jnp.float32),
            pltpu.VMEM((CHUNK,), jnp.int32),
            pltpu.VMEM((CHUNK,), jnp.int32),
            pltpu.VMEM((CHUNK, D), jnp.float32),
            pltpu.VMEM((ZR, D), jnp.float32),
            pltpu.SemaphoreType.DMA,
        ],
    )
    def segsum(table, src, dst, out, acc, sidx, didx, rows, zbuf, sem):
        c = lax.axis_index("c")
        s = lax.axis_index("s")

        def zero_row(i, carry):
            for j in range(D // 16):
                zbuf[i, pl.ds(j * 16, 16)] = jnp.zeros((16,), jnp.float32)
            return carry

        lax.fori_loop(0, ZR, zero_row, 0)
        row0 = s * ROWS_PER_TILE
        for k in range(N_COPIES):
            pltpu.sync_copy(zbuf, acc.at[pl.ds(row0 + k * ZR, ZR)])
        plsc.subcore_barrier()

        def body(j, carry):
            cid = s + j * NS

            @pl.when(cid < CHUNKS_PER_CORE)
            def _():
                e0 = c * (E // NC) + cid * CHUNK
                pltpu.sync_copy(src.at[pl.ds(e0, CHUNK)], sidx)
                pltpu.sync_copy(dst.at[pl.ds(e0, CHUNK)], didx)
                pltpu.async_copy(table.at[sidx], rows, sem).wait()
                pltpu.sync_copy(rows, acc.at[didx], add=True)

            return carry

        lax.fori_loop(0, ITERS, body, 0)
        plsc.subcore_barrier()

        for k in range(N_COPIES):
            r = row0 + k * ZR
            pltpu.sync_copy(acc.at[pl.ds(r, ZR)], zbuf)
            pltpu.sync_copy(zbuf, out.at[pl.ds(c * NP + r, ZR)])

    return segsum


_segsum144 = _make_segsum(D_IN + 16)
_segsum128 = _make_segsum(D_OUT)

_R = 1000


def _dense0_body(x_ref, a0_ref, a1_ref, ws0_ref, wn0_ref, b0_ref, wn1_ref,
                 h_ref, p_ref):
    agg = a0_ref[:, :D_IN] + a1_ref[:, :D_IN]
    deg = a0_ref[:, D_IN:D_IN + 16] + a1_ref[:, D_IN:D_IN + 16]
    invd = 1.0 / jnp.clip(deg[:, :1], 1.0, None)
    nb = agg * invd
    h = x_ref[...] @ ws0_ref[...] + nb @ wn0_ref[...] + b0_ref[...]
    h = jnp.maximum(h, 0.0)
    h_ref[...] = h
    p_ref[...] = h @ wn1_ref[...]


def _dense0(x, a0, a1, Ws0, Wn0, b0, Wn1):
    D0 = D_IN + 16
    return pl.pallas_call(
        _dense0_body,
        grid=(N // _R,),
        in_specs=[
            pl.BlockSpec((_R, D_IN), lambda i: (i, 0)),
            pl.BlockSpec((_R, D0), lambda i: (i, 0)),
            pl.BlockSpec((_R, D0), lambda i: (i, 0)),
            pl.BlockSpec((D_IN, D_HID), lambda i: (0, 0)),
            pl.BlockSpec((D_IN, D_HID), lambda i: (0, 0)),
            pl.BlockSpec((1, D_HID), lambda i: (0, 0)),
            pl.BlockSpec((D_HID, D_OUT), lambda i: (0, 0)),
        ],
        out_specs=[
            pl.BlockSpec((_R, D_HID), lambda i: (i, 0)),
            pl.BlockSpec((_R, D_OUT), lambda i: (i, 0)),
        ],
        out_shape=[
            jax.ShapeDtypeStruct((N, D_HID), jnp.float32),
            jax.ShapeDtypeStruct((N, D_OUT), jnp.float32),
        ],
    )(x, a0, a1, Ws0, Wn0, b0, Wn1)


def _dense1_body(h_ref, a0_ref, a1_ref, d0_ref, d1_ref, ws1_ref, b1_ref,
                 o_ref):
    deg = d0_ref[:, :1] + d1_ref[:, :1]
    invd = 1.0 / jnp.clip(deg, 1.0, None)
    nb = (a0_ref[...] + a1_ref[...]) * invd
    o_ref[...] = h_ref[...] @ ws1_ref[...] + nb + b1_ref[...]


def _dense1(h, a0, a1, d0, d1, Ws1, b1):
    return pl.pallas_call(
        _dense1_body,
        grid=(N // _R,),
        in_specs=[
            pl.BlockSpec((_R, D_HID), lambda i: (i, 0)),
            pl.BlockSpec((_R, D_OUT), lambda i: (i, 0)),
            pl.BlockSpec((_R, D_OUT), lambda i: (i, 0)),
            pl.BlockSpec((_R, 16), lambda i: (i, 0)),
            pl.BlockSpec((_R, 16), lambda i: (i, 0)),
            pl.BlockSpec((D_HID, D_OUT), lambda i: (0, 0)),
            pl.BlockSpec((1, D_OUT), lambda i: (0, 0)),
        ],
        out_specs=pl.BlockSpec((_R, D_OUT), lambda i: (i, 0)),
        out_shape=jax.ShapeDtypeStruct((N, D_OUT), jnp.float32),
    )(h, a0, a1, d0, d1, Ws1, b1)


def kernel(x, edge_index, W_self0, W_neigh0, b0, W_self1, W_neigh1, b1):
    src = edge_index[0]
    dst = edge_index[1]
    x_ext = jnp.concatenate(
        [x, jnp.ones((N, 16), jnp.float32)], axis=1)
    parts0 = _segsum144(x_ext, src, dst)
    a0, a1 = parts0[:N], parts0[NP:NP + N]
    h, p = _dense0(x, a0, a1, W_self0, W_neigh0,
                   b0.reshape(1, -1), W_neigh1)
    parts1 = _segsum128(p, src, dst)
    out = _dense1(h, parts1[:N], parts1[NP:NP + N],
                  a0[:, D_IN:D_IN + 16], a1[:, D_IN:D_IN + 16],
                  W_self1, b1.reshape(1, -1))
    return out
